# Initial kernel scaffold; baseline (speedup 1.0000x reference)
import jax
import jax.numpy as jnp
from jax.experimental import pallas as pl


def kernel(features, edge_coarse, edge_fine):
    raise NotImplementedError("write your pallas kernel here")


# speedup vs baseline: 5.6993x; 5.6993x over previous
import jax
import jax.numpy as jnp
from jax import lax
from jax.experimental import pallas as pl
from jax.experimental.pallas import tpu as pltpu
from jax.experimental.pallas import tpu_sc as plsc

NCOARSE = 10000
NFINE = 50000
NEDGE = 320000
D = 128

NC = 2
NS = 16
L = 16

C = 12
S = 4480
ACC_ROWS = S + 8
K = 128
RPT = S // NS
RB = 56


def _body(feat_hbm, ec_hbm, ef_hbm, starts_hbm, out_hbm,
          svec, finebuf, coarsebuf, localbuf, gbuf, ones,
          rowbuf, cntbuf, acc, cnt, sem):
    cid = lax.axis_index("c")
    sid = lax.axis_index("s")
    iota = lax.iota(jnp.int32, L)
    zrow = jnp.zeros((L,), jnp.float32)
    orow = jnp.ones((L,), jnp.float32)

    pltpu.sync_copy(starts_hbm, svec)

    def init_ones(i, _):
        for j in range(D // L):
            ones[i, pl.ds(j * L, L)] = orow
        return 0
    lax.fori_loop(0, K, init_ones, 0)

    for p in range(C // NC):
        cbase = (cid + NC * p) * S

        plsc.subcore_barrier()

        def zero_row(i, _):
            for j in range(D // L):
                rowbuf[i, pl.ds(j * L, L)] = zrow
                cntbuf[i, pl.ds(j * L, L)] = zrow
            return 0
        lax.fori_loop(0, RB, zero_row, 0)

        for b in range(RPT // RB):
            r0 = pl.multiple_of(sid * RPT + b * RB, 8)
            pltpu.sync_copy(rowbuf, acc.at[pl.ds(r0, RB)])
            pltpu.sync_copy(cntbuf, cnt.at[pl.ds(r0, RB)])

        @pl.when(sid == 0)
        def _():
            pltpu.sync_copy(rowbuf.at[pl.ds(0, ACC_ROWS - S)],
                            acc.at[pl.ds(S, ACC_ROWS - S)])
            pltpu.sync_copy(cntbuf.at[pl.ds(0, ACC_ROWS - S)],
                            cnt.at[pl.ds(S, ACC_ROWS - S)])

        plsc.subcore_barrier()

        sv = svec[pl.ds(0, L)]
        start_c = jnp.where(cid == 0, sv[NC * p], sv[NC * p + 1])
        end_c = jnp.where(cid == 0, sv[NC * p + 1], sv[NC * p + 2])

        n = end_c - start_c
        start_t = start_c + lax.shift_right_logical(sid * n, 4)
        end_t = start_c + lax.shift_right_logical((sid + 1) * n, 4)
        a_t = jnp.bitwise_and(start_t, -8)
        nb = lax.shift_right_logical(end_t - a_t + (K - 1), 7)

        def block(b, _):
            e0 = pl.multiple_of(a_t + b * K, 8)
            pltpu.sync_copy(ef_hbm.at[pl.ds(e0, K)], finebuf)
            pltpu.sync_copy(ec_hbm.at[pl.ds(e0, K)], coarsebuf)
            for i in range(K // L):
                fv = finebuf[pl.ds(i * L, L)]
                ev = iota + (e0 + i * L)
                valid = (ev >= start_t) & (ev < end_t)
                lv = jnp.where(valid, fv - cbase, S)
                localbuf[pl.ds(i * L, L)] = lv
            pltpu.async_copy(feat_hbm.at[coarsebuf], gbuf, sem).wait()
            pltpu.sync_copy(gbuf, acc.at[localbuf], add=True)
            pltpu.sync_copy(ones, cnt.at[localbuf], add=True)
            return 0
        lax.fori_loop(0, nb, block, 0)

        plsc.subcore_barrier()

        for b in range(RPT // RB):
            row0 = cbase + sid * RPT + b * RB

            @pl.when(row0 < NFINE)
            def _():
                w0 = pl.multiple_of(jnp.minimum(row0, NFINE - RB), 8)
                l0 = pl.multiple_of(w0 - cbase, 8)
                pltpu.sync_copy(acc.at[pl.ds(l0, RB)], rowbuf)
                pltpu.sync_copy(cnt.at[pl.ds(l0, RB)], cntbuf)

                def frow(r, _):
                    cv = cntbuf[r, pl.ds(0, L)]
                    rec = 1.0 / jnp.maximum(cv, 1.0)
                    for j in range(D // L):
                        rowbuf[r, pl.ds(j * L, L)] = (
                            rowbuf[r, pl.ds(j * L, L)] * rec)
                    return 0
                lax.fori_loop(0, RB, frow, 0)
                pltpu.sync_copy(rowbuf, out_hbm.at[pl.ds(w0, RB)])


def kernel(features, edge_coarse, edge_fine):
    ec = jnp.concatenate(
        [edge_coarse, jnp.zeros((K,), jnp.int32)])
    ef = jnp.concatenate(
        [edge_fine, jnp.full((K,), NFINE, jnp.int32)])

    bnds = jnp.arange(1, C, dtype=jnp.int32) * S
    si = jnp.searchsorted(edge_fine, bnds).astype(jnp.int32)
    starts = jnp.concatenate([
        jnp.zeros((1,), jnp.int32), si,
        jnp.full((2 * L - C, ), NEDGE, jnp.int32)])

    mesh = plsc.VectorSubcoreMesh(core_axis_name="c", subcore_axis_name="s")
    run = pl.kernel(
        _body,
        out_type=jax.ShapeDtypeStruct((NFINE, D), jnp.float32),
        mesh=mesh,
        scratch_types=[
            pltpu.VMEM((2 * L,), jnp.int32),
            pltpu.VMEM((K,), jnp.int32),
            pltpu.VMEM((K,), jnp.int32),
            pltpu.VMEM((K,), jnp.int32),
            pltpu.VMEM((K, D), jnp.float32),
            pltpu.VMEM((K, D), jnp.float32),
            pltpu.VMEM((RB, D), jnp.float32),
            pltpu.VMEM((RB, D), jnp.float32),
            pltpu.VMEM_SHARED((ACC_ROWS, D), jnp.float32),
            pltpu.VMEM_SHARED((ACC_ROWS, D), jnp.float32),
            pltpu.SemaphoreType.DMA,
        ],
    )
    return run(features, ec, ef, starts)


# speedup vs baseline: 5.9801x; 1.0493x over previous
import jax
import jax.numpy as jnp
from jax import lax
from jax.experimental import pallas as pl
from jax.experimental.pallas import tpu as pltpu
from jax.experimental.pallas import tpu_sc as plsc

NCOARSE = 10000
NFINE = 50000
NEDGE = 320000
D = 128

NC = 2
NS = 16
L = 16

C = 12
S = 4480
SP = 4608
ACC_ROWS = S + 8
K = 128
FB = 128
NBLK = S // FB
OUTR = C * S


def _body(feat_hbm, ec_hbm, ef_hbm, starts_hbm, out_hbm,
          svec, finebuf, coarsebuf, localbuf, gbuf,
          rowbuf, cvec, cntbuf, recbuf, acc, cnt16, sem):
    cid = lax.axis_index("c")
    sid = lax.axis_index("s")
    iota = lax.iota(jnp.int32, L)
    zrow = jnp.zeros((L,), jnp.float32)
    orow = jnp.ones((L,), jnp.float32)

    pltpu.sync_copy(starts_hbm, svec)

    for p in range(C // NC):
        cbase = (cid + NC * p) * S

        plsc.subcore_barrier()

        def zero_row(i, _):
            for j in range(D // L):
                rowbuf[i, pl.ds(j * L, L)] = zrow
            return 0
        lax.fori_loop(0, FB, zero_row, 0)

        for bb in range(pl.cdiv(NBLK, NS)):
            b = sid + NS * bb

            @pl.when(b < NBLK)
            def _():
                r0 = pl.multiple_of(b * FB, 8)
                pltpu.sync_copy(rowbuf, acc.at[pl.ds(r0, FB)])

        @pl.when(sid == 0)
        def _():
            pltpu.sync_copy(rowbuf.at[pl.ds(0, ACC_ROWS - S)],
                            acc.at[pl.ds(S, ACC_ROWS - S)])

        def zero_cnt(i, _):
            cvec[pl.ds(i * L, L)] = zrow
            return 0
        lax.fori_loop(0, SP // L, zero_cnt, 0)

        plsc.subcore_barrier()

        sv = svec[pl.ds(0, L)]
        start_c = jnp.where(cid == 0, sv[NC * p], sv[NC * p + 1])
        end_c = jnp.where(cid == 0, sv[NC * p + 1], sv[NC * p + 2])

        n = end_c - start_c
        start_t = start_c + lax.shift_right_logical(sid * n, 4)
        end_t = start_c + lax.shift_right_logical((sid + 1) * n, 4)
        a_t = jnp.bitwise_and(start_t, -8)
        nb = lax.shift_right_logical(end_t - a_t + (K - 1), 7)

        def block(b, _):
            e0 = pl.multiple_of(a_t + b * K, 8)
            pltpu.sync_copy(ef_hbm.at[pl.ds(e0, K)], finebuf)
            pltpu.sync_copy(ec_hbm.at[pl.ds(e0, K)], coarsebuf)
            for i in range(K // L):
                fv = finebuf[pl.ds(i * L, L)]
                ev = iota + (e0 + i * L)
                valid = (ev >= start_t) & (ev < end_t)
                lv = jnp.where(valid, fv - cbase, S)
                localbuf[pl.ds(i * L, L)] = lv
                plsc.addupdate_scatter(cvec, [lv], orow)
            pltpu.async_copy(feat_hbm.at[coarsebuf], gbuf, sem).wait()
            pltpu.sync_copy(gbuf, acc.at[localbuf], add=True)
            return 0
        lax.fori_loop(0, nb, block, 0)

        pltpu.sync_copy(cvec, cnt16.at[sid])

        plsc.subcore_barrier()

        for bb in range(pl.cdiv(NBLK, NS)):
            b = sid + NS * bb

            @pl.when(b < NBLK)
            def _():
                r0 = pl.multiple_of(b * FB, 128)
                pltpu.sync_copy(acc.at[pl.ds(r0, FB)], rowbuf)
                pltpu.sync_copy(cnt16.at[:, pl.ds(r0, FB)], cntbuf)

                for j in range(FB // L):
                    cs = zrow
                    for t in range(NS):
                        cs = cs + cntbuf[t, pl.ds(j * L, L)]
                    recbuf[pl.ds(j * L, L)] = 1.0 / jnp.maximum(cs, 1.0)

                def frow(r, _):
                    rec = plsc.load_gather(
                        recbuf, [jnp.full((L,), r, jnp.int32)])
                    for j in range(D // L):
                        rowbuf[r, pl.ds(j * L, L)] = (
                            rowbuf[r, pl.ds(j * L, L)] * rec)
                    return 0
                lax.fori_loop(0, FB, frow, 0)
                w0 = pl.multiple_of(cbase + r0, 128)
                pltpu.sync_copy(rowbuf, out_hbm.at[pl.ds(w0, FB)])


def kernel(features, edge_coarse, edge_fine):
    ec = jnp.concatenate(
        [edge_coarse, jnp.zeros((K,), jnp.int32)])
    ef = jnp.concatenate(
        [edge_fine, jnp.full((K,), NFINE, jnp.int32)])

    bnds = jnp.arange(1, C, dtype=jnp.int32) * S
    si = jnp.searchsorted(edge_fine, bnds).astype(jnp.int32)
    starts = jnp.concatenate([
        jnp.zeros((1,), jnp.int32), si,
        jnp.full((2 * L - C, ), NEDGE, jnp.int32)])

    mesh = plsc.VectorSubcoreMesh(core_axis_name="c", subcore_axis_name="s")
    run = pl.kernel(
        _body,
        out_type=jax.ShapeDtypeStruct((OUTR, D), jnp.float32),
        mesh=mesh,
        compiler_params=pltpu.CompilerParams(needs_layout_passes=False),
        scratch_types=[
            pltpu.VMEM((2 * L,), jnp.int32),
            pltpu.VMEM((K,), jnp.int32),
            pltpu.VMEM((K,), jnp.int32),
            pltpu.VMEM((K,), jnp.int32),
            pltpu.VMEM((K, D), jnp.float32),
            pltpu.VMEM((FB, D), jnp.float32),
            pltpu.VMEM((SP,), jnp.float32),
            pltpu.VMEM((NS, FB), jnp.float32),
            pltpu.VMEM((FB,), jnp.float32),
            pltpu.VMEM_SHARED((ACC_ROWS, D), jnp.float32),
            pltpu.VMEM_SHARED((NS, SP), jnp.float32),
            pltpu.SemaphoreType.DMA,
        ],
    )
    out = run(features, ec, ef, starts)
    return out[:NFINE]


# speedup vs baseline: 8.0007x; 1.3379x over previous
import jax
import jax.numpy as jnp
from jax import lax
from jax.experimental import pallas as pl
from jax.experimental.pallas import tpu as pltpu
from jax.experimental.pallas import tpu_sc as plsc

NCOARSE = 10000
NFINE = 50000
NEDGE = 320000
D = 128

NC = 2
NS = 16
L = 16

C = 12
S = 4480
SP = 4608
ACC_ROWS = S + 8
K = 128
FB = 128
NBLK = S // FB
OUTR = C * S


def _body(feat_hbm, ec_hbm, ef_hbm, starts_hbm, out_hbm,
          svec, finebuf, coarsebuf, localbuf, gbuf,
          finebuf2, coarsebuf2, localbuf2, gbuf2,
          rowbuf, cvec, cntbuf, recbuf, acc, cnt16, sem, sem2):
    cid = lax.axis_index("c")
    sid = lax.axis_index("s")
    iota = lax.iota(jnp.int32, L)
    zrow = jnp.zeros((L,), jnp.float32)
    orow = jnp.ones((L,), jnp.float32)

    pltpu.sync_copy(starts_hbm, svec)

    for p in range(C // NC):
        cbase = (cid + NC * p) * S

        plsc.subcore_barrier()

        def zero_row(i, _):
            for j in range(D // L):
                rowbuf[i, pl.ds(j * L, L)] = zrow
            return 0
        lax.fori_loop(0, FB, zero_row, 0)

        for bb in range(pl.cdiv(NBLK, NS)):
            b = sid + NS * bb

            @pl.when(b < NBLK)
            def _():
                r0 = pl.multiple_of(b * FB, 8)
                pltpu.sync_copy(rowbuf, acc.at[pl.ds(r0, FB)])

        @pl.when(sid == 0)
        def _():
            pltpu.sync_copy(rowbuf.at[pl.ds(0, ACC_ROWS - S)],
                            acc.at[pl.ds(S, ACC_ROWS - S)])

        def zero_cnt(i, _):
            cvec[pl.ds(i * L, L)] = zrow
            return 0
        lax.fori_loop(0, SP // L, zero_cnt, 0)

        plsc.subcore_barrier()

        sv = svec[pl.ds(0, L)]
        start_c = jnp.where(cid == 0, sv[NC * p], sv[NC * p + 1])
        end_c = jnp.where(cid == 0, sv[NC * p + 1], sv[NC * p + 2])

        n = end_c - start_c
        start_t = start_c + lax.shift_right_logical(sid * n, 4)
        end_t = start_c + lax.shift_right_logical((sid + 1) * n, 4)
        a_t = jnp.bitwise_and(start_t, -8)
        nb = lax.shift_right_logical(end_t - a_t + (K - 1), 7)

        fbufs = (finebuf, finebuf2)
        cbufs = (coarsebuf, coarsebuf2)
        lbufs = (localbuf, localbuf2)
        gbufs = (gbuf, gbuf2)
        sems = (sem, sem2)

        def prep(b, q):
            e0 = pl.multiple_of(a_t + b * K, 8)
            pltpu.sync_copy(ef_hbm.at[pl.ds(e0, K)], fbufs[q])
            pltpu.sync_copy(ec_hbm.at[pl.ds(e0, K)], cbufs[q])
            for i in range(K // L):
                fv = fbufs[q][pl.ds(i * L, L)]
                ev = iota + (e0 + i * L)
                valid = (ev >= start_t) & (ev < end_t)
                lv = jnp.where(valid, fv - cbase, S)
                lbufs[q][pl.ds(i * L, L)] = lv
                plsc.addupdate_scatter(cvec, [lv], orow)
            pltpu.async_copy(feat_hbm.at[cbufs[q]], gbufs[q], sems[q])

        def drain_scatter(q):
            pltpu.make_async_copy(
                feat_hbm.at[cbufs[q]], gbufs[q], sems[q]).wait()
            pltpu.sync_copy(gbufs[q], acc.at[lbufs[q]], add=True)

        @pl.when(nb > 0)
        def _():
            prep(0, 0)

        def pair(g, _):
            b1 = 2 * g + 1
            b2 = 2 * g + 2

            @pl.when(b1 < nb)
            def _():
                prep(b1, 1)
            drain_scatter(0)

            @pl.when(b2 < nb)
            def _():
                prep(b2, 0)

            @pl.when(b1 < nb)
            def _():
                drain_scatter(1)
            return 0
        lax.fori_loop(0, lax.shift_right_logical(nb + 1, 1), pair, 0)

        pltpu.sync_copy(cvec, cnt16.at[sid])

        plsc.subcore_barrier()

        for bb in range(pl.cdiv(NBLK, NS)):
            b = sid + NS * bb

            @pl.when(b < NBLK)
            def _():
                r0 = pl.multiple_of(b * FB, 128)
                pltpu.sync_copy(acc.at[pl.ds(r0, FB)], rowbuf)
                pltpu.sync_copy(cnt16.at[:, pl.ds(r0, FB)], cntbuf)

                for j in range(FB // L):
                    cs = zrow
                    for t in range(NS):
                        cs = cs + cntbuf[t, pl.ds(j * L, L)]
                    recbuf[pl.ds(j * L, L)] = 1.0 / jnp.maximum(cs, 1.0)

                def frow(r, _):
                    rec = plsc.load_gather(
                        recbuf, [jnp.full((L,), r, jnp.int32)])
                    for j in range(D // L):
                        rowbuf[r, pl.ds(j * L, L)] = (
                            rowbuf[r, pl.ds(j * L, L)] * rec)
                    return 0
                lax.fori_loop(0, FB, frow, 0)
                w0 = pl.multiple_of(cbase + r0, 128)
                pltpu.sync_copy(rowbuf, out_hbm.at[pl.ds(w0, FB)])


def kernel(features, edge_coarse, edge_fine):
    ec = jnp.concatenate(
        [edge_coarse, jnp.zeros((K,), jnp.int32)])
    ef = jnp.concatenate(
        [edge_fine, jnp.full((K,), NFINE, jnp.int32)])

    bnds = jnp.arange(1, C, dtype=jnp.int32) * S
    si = jnp.searchsorted(edge_fine, bnds).astype(jnp.int32)
    starts = jnp.concatenate([
        jnp.zeros((1,), jnp.int32), si,
        jnp.full((2 * L - C, ), NEDGE, jnp.int32)])

    mesh = plsc.VectorSubcoreMesh(core_axis_name="c", subcore_axis_name="s")
    run = pl.kernel(
        _body,
        out_type=jax.ShapeDtypeStruct((OUTR, D), jnp.float32),
        mesh=mesh,
        compiler_params=pltpu.CompilerParams(needs_layout_passes=False),
        scratch_types=[
            pltpu.VMEM((2 * L,), jnp.int32),
            pltpu.VMEM((K,), jnp.int32),
            pltpu.VMEM((K,), jnp.int32),
            pltpu.VMEM((K,), jnp.int32),
            pltpu.VMEM((K, D), jnp.float32),
            pltpu.VMEM((K,), jnp.int32),
            pltpu.VMEM((K,), jnp.int32),
            pltpu.VMEM((K,), jnp.int32),
            pltpu.VMEM((K, D), jnp.float32),
            pltpu.VMEM((FB, D), jnp.float32),
            pltpu.VMEM((SP,), jnp.float32),
            pltpu.VMEM((NS, FB), jnp.float32),
            pltpu.VMEM((FB,), jnp.float32),
            pltpu.VMEM_SHARED((ACC_ROWS, D), jnp.float32),
            pltpu.VMEM_SHARED((NS, SP), jnp.float32),
            pltpu.SemaphoreType.DMA,
            pltpu.SemaphoreType.DMA,
        ],
    )
    out = run(features, ec, ef, starts)
    return out[:NFINE]
